# Initial kernel scaffold; baseline (speedup 1.0000x reference)
import jax
import jax.numpy as jnp
from jax.experimental import pallas as pl


def kernel(images):
    raise NotImplementedError("write your pallas kernel here")


# speedup vs baseline: 61.2704x; 61.2704x over previous
import jax
import jax.numpy as jnp
from jax.experimental import pallas as pl
from jax.experimental.pallas import tpu as pltpu

_TOK = 16


def _ring_positions():
    pos = [(i, j) for i in (0, 1) for j in range(8)]
    for i in range(2, 6):
        pos += [(i, 0), (i, 1), (i, 6), (i, 7)]
    pos += [(i, j) for i in (6, 7) for j in range(8)]
    return pos


_FULL = [(i, j) for i in range(8) for j in range(8)]
_RING = _ring_positions()


def _pool_matrix(rows, pool):
    k = jax.lax.broadcasted_iota(jnp.int32, (rows, rows // pool), 0)
    v = jax.lax.broadcasted_iota(jnp.int32, (rows, rows // pool), 1)
    return jnp.where(k // pool == v, 1.0, 0.0).astype(jnp.float32)


def _fov_kernel(img_ref, out_ref, s1_ref, s2_ref):
    img = img_ref[0, 0]
    p0 = img[192:320, 192:320]
    s1_ref[...] = jnp.dot(img[128:384, 128:384], _pool_matrix(256, 2),
                          preferred_element_type=jnp.float32)
    p1 = (s1_ref[0::2, :] + s1_ref[1::2, :]) * 0.25
    s2_ref[...] = jnp.dot(img, _pool_matrix(512, 4),
                          preferred_element_type=jnp.float32)
    p2 = (s2_ref[0::4, :] + s2_ref[1::4, :]
          + s2_ref[2::4, :] + s2_ref[3::4, :]) * 0.0625
    n = 0
    for p, positions in ((p0, _FULL), (p1, _RING), (p2, _RING)):
        for (i, j) in positions:
            out_ref[0, n, 0] = p[16 * i:16 * i + 16, 16 * j:16 * j + 16]
            n += 1


def kernel(images):
    B, C, H, W = images.shape
    return pl.pallas_call(
        _fov_kernel,
        grid=(B, C),
        in_specs=[pl.BlockSpec((1, 1, H, W), lambda b, c: (b, c, 0, 0))],
        out_specs=pl.BlockSpec((1, 160, 1, _TOK, _TOK),
                               lambda b, c: (b, 0, c, 0, 0)),
        out_shape=jax.ShapeDtypeStruct((B, 160, C, _TOK, _TOK), jnp.float32),
        scratch_shapes=[pltpu.VMEM((256, 128), jnp.float32),
                        pltpu.VMEM((512, 128), jnp.float32)],
    )(images)


# speedup vs baseline: 81.8370x; 1.3357x over previous
import jax
import jax.numpy as jnp
from jax.experimental import pallas as pl
from jax.experimental.pallas import tpu as pltpu

_TOK = 16


def _pool_matrix(rows, pool):
    k = jax.lax.broadcasted_iota(jnp.int32, (rows, rows // pool), 0)
    v = jax.lax.broadcasted_iota(jnp.int32, (rows, rows // pool), 1)
    return jnp.where(k // pool == v, 1.0, 0.0).astype(jnp.float32)


def _tokens(p):
    x = p.reshape(8, _TOK, 8, _TOK)
    x = jnp.transpose(x, (0, 2, 1, 3))
    return x.reshape(64, 16 * _TOK)


def _ring(x):
    parts = [x[0:16]]
    for i in range(2, 6):
        parts.append(x[8 * i:8 * i + 2])
        parts.append(x[8 * i + 6:8 * i + 8])
    parts.append(x[48:64])
    return jnp.concatenate(parts, axis=0)


def _fov_kernel(img_ref, out_ref, s1_ref, s2_ref):
    c = pl.program_id(1)
    p0 = img_ref[0, 0, 192:320, 192:320]
    s1_ref[...] = jnp.dot(img_ref[0, 0, 128:384, 128:384], _pool_matrix(256, 2),
                          preferred_element_type=jnp.float32)
    p1 = (s1_ref[0::2, :] + s1_ref[1::2, :]) * 0.25
    s2_ref[...] = jnp.dot(img_ref[0, 0], _pool_matrix(512, 4),
                          preferred_element_type=jnp.float32)
    p2 = (s2_ref[0::4, :] + s2_ref[1::4, :]
          + s2_ref[2::4, :] + s2_ref[3::4, :]) * 0.0625
    block = jnp.concatenate(
        [_tokens(p0), _ring(_tokens(p1)), _ring(_tokens(p2))], axis=0)
    for ci in range(3):
        @pl.when(c == ci)
        def _():
            out_ref[0, :, 256 * ci:256 * ci + 256] = block


def kernel(images):
    B, C, H, W = images.shape
    out = pl.pallas_call(
        _fov_kernel,
        grid=(B, C),
        in_specs=[pl.BlockSpec((1, 1, H, W), lambda b, c: (b, c, 0, 0))],
        out_specs=pl.BlockSpec((1, 160, C * 256), lambda b, c: (b, 0, 0)),
        out_shape=jax.ShapeDtypeStruct((B, 160, C * 256), jnp.float32),
        scratch_shapes=[pltpu.VMEM((256, 128), jnp.float32),
                        pltpu.VMEM((512, 128), jnp.float32)],
    )(images)
    return out.reshape(B, 160, C, _TOK, _TOK)


# speedup vs baseline: 82.0070x; 1.0021x over previous
import jax
import jax.numpy as jnp
from jax.experimental import pallas as pl
from jax.experimental.pallas import tpu as pltpu

_TOK = 16


def _pool_matrix(rows, pool):
    k = jax.lax.broadcasted_iota(jnp.int32, (rows, rows // pool), 0)
    v = jax.lax.broadcasted_iota(jnp.int32, (rows, rows // pool), 1)
    return jnp.where(k // pool == v, 1.0, 0.0).astype(jnp.float32)


def _tokens(p):
    x = p.reshape(8, _TOK, 8, _TOK)
    x = jnp.transpose(x, (0, 2, 1, 3))
    return x.reshape(64, 16 * _TOK)


def _ring(x):
    parts = [x[0:16]]
    for i in range(2, 6):
        parts.append(x[8 * i:8 * i + 2])
        parts.append(x[8 * i + 6:8 * i + 8])
    parts.append(x[48:64])
    return jnp.concatenate(parts, axis=0)


def _fov_kernel(img_ref, out_ref, s1_ref, s2_ref):
    c = pl.program_id(1)
    p0 = img_ref[0, 0, 192:320, 192:320]
    s1_ref[...] = jnp.dot(img_ref[0, 0, 128:384, 128:384], _pool_matrix(256, 2),
                          preferred_element_type=jnp.float32)
    p1 = (s1_ref[0::2, :] + s1_ref[1::2, :]) * 0.25
    s2_ref[...] = jnp.dot(img_ref[0, 0], _pool_matrix(512, 4),
                          preferred_element_type=jnp.float32)
    p2 = (s2_ref[0::4, :] + s2_ref[1::4, :]
          + s2_ref[2::4, :] + s2_ref[3::4, :]) * 0.0625
    block = jnp.concatenate(
        [_tokens(p0), _ring(_tokens(p1)), _ring(_tokens(p2))], axis=0)
    for ci in range(3):
        @pl.when(c == ci)
        def _():
            out_ref[0, :, 256 * ci:256 * ci + 256] = block


def kernel(images):
    B, C, H, W = images.shape
    out = pl.pallas_call(
        _fov_kernel,
        grid=(B, C),
        in_specs=[pl.BlockSpec((1, 1, H, W), lambda b, c: (b, c, 0, 0))],
        out_specs=pl.BlockSpec((1, 160, C * 256), lambda b, c: (b, 0, 0)),
        out_shape=jax.ShapeDtypeStruct((B, 160, C * 256), jnp.float32),
        scratch_shapes=[pltpu.VMEM((256, 128), jnp.float32),
                        pltpu.VMEM((512, 128), jnp.float32)],
        compiler_params=pltpu.CompilerParams(
            dimension_semantics=("parallel", "arbitrary")),
    )(images)
    return out.reshape(B, 160, C, _TOK, _TOK)


# speedup vs baseline: 92.6838x; 1.1302x over previous
import jax
import jax.numpy as jnp
from jax.experimental import pallas as pl
from jax.experimental.pallas import tpu as pltpu

_TOK = 16


def _pool_matrix(rows, pool):
    k = jax.lax.broadcasted_iota(jnp.int32, (rows, rows // pool), 0)
    v = jax.lax.broadcasted_iota(jnp.int32, (rows, rows // pool), 1)
    return jnp.where(k // pool == v, 1.0, 0.0).astype(jnp.float32)


def _tokens(p):
    x = p.reshape(8, _TOK, 8, _TOK)
    x = jnp.transpose(x, (0, 2, 1, 3))
    return x.reshape(64, 16 * _TOK)


def _ring(x):
    parts = [x[0:16]]
    for i in range(2, 6):
        parts.append(x[8 * i:8 * i + 2])
        parts.append(x[8 * i + 6:8 * i + 8])
    parts.append(x[48:64])
    return jnp.concatenate(parts, axis=0)


def _fov_kernel(img_ref, out_ref, s1_ref, s2_ref):
    cols = []
    for c in range(3):
        p0 = img_ref[0, c, 192:320, 192:320]
        s1_ref[...] = jnp.dot(img_ref[0, c, 128:384, 128:384],
                              _pool_matrix(256, 2),
                              preferred_element_type=jnp.float32)
        p1 = (s1_ref[0::2, :] + s1_ref[1::2, :]) * 0.25
        s2_ref[...] = jnp.dot(img_ref[0, c], _pool_matrix(512, 4),
                              preferred_element_type=jnp.float32)
        p2 = (s2_ref[0::4, :] + s2_ref[1::4, :]
              + s2_ref[2::4, :] + s2_ref[3::4, :]) * 0.0625
        cols.append(jnp.concatenate(
            [_tokens(p0), _ring(_tokens(p1)), _ring(_tokens(p2))], axis=0))
    out_ref[0] = jnp.concatenate(cols, axis=1)


def kernel(images):
    B, C, H, W = images.shape
    out = pl.pallas_call(
        _fov_kernel,
        grid=(B,),
        in_specs=[pl.BlockSpec((1, C, H, W), lambda b: (b, 0, 0, 0))],
        out_specs=pl.BlockSpec((1, 160, C * 256), lambda b: (b, 0, 0)),
        out_shape=jax.ShapeDtypeStruct((B, 160, C * 256), jnp.float32),
        scratch_shapes=[pltpu.VMEM((256, 128), jnp.float32),
                        pltpu.VMEM((512, 128), jnp.float32)],
        compiler_params=pltpu.CompilerParams(
            dimension_semantics=("arbitrary",)),
    )(images)
    return out.reshape(B, 160, C, _TOK, _TOK)


# speedup vs baseline: 153.8277x; 1.6597x over previous
import jax
import jax.numpy as jnp
from jax.experimental import pallas as pl
from jax.experimental.pallas import tpu as pltpu

_TOK = 16


def _pool_matrix(rows, pool):
    k = jax.lax.broadcasted_iota(jnp.int32, (rows, rows // pool), 0)
    v = jax.lax.broadcasted_iota(jnp.int32, (rows, rows // pool), 1)
    return jnp.where(k // pool == v, 1.0, 0.0).astype(jnp.float32)


def _roll_sub(x, m):
    r = x.reshape(16, 8, 128)
    r = jnp.concatenate([r[:, 8 - m:], r[:, :8 - m]], axis=1)
    return r.reshape(128, 128)


def _gran_xpose(x):
    s = jax.lax.broadcasted_iota(jnp.int32, (128, 128), 0) % 8
    g = jax.lax.broadcasted_iota(jnp.int32, (128, 128), 1) // _TOK
    for m in (4, 2, 1):
        xa = jnp.roll(_roll_sub(x, m), -_TOK * m, axis=1)
        xb = jnp.roll(_roll_sub(x, 8 - m), _TOK * m, axis=1)
        sm = (s & m) != 0
        gm = (g & m) != 0
        x = jnp.where(sm == gm, x, jnp.where(sm, xa, xb))
    return x


def _tokens(p):
    q = _gran_xpose(p).reshape(8, 2, 8, 128)
    return jnp.concatenate(
        [q[:, 0].reshape(64, 128), q[:, 1].reshape(64, 128)], axis=1)


def _ring(x):
    parts = [x[0:16]]
    for i in range(2, 6):
        parts.append(x[8 * i:8 * i + 2])
        parts.append(x[8 * i + 6:8 * i + 8])
    parts.append(x[48:64])
    return jnp.concatenate(parts, axis=0)


def _fov_kernel(img_ref, out_ref, s1_ref, s2_ref):
    cols = []
    for c in range(3):
        p0 = img_ref[0, c, 192:320, 192:320]
        s1_ref[...] = jnp.dot(img_ref[0, c, 128:384, 128:384],
                              _pool_matrix(256, 2),
                              preferred_element_type=jnp.float32)
        p1 = (s1_ref[0::2, :] + s1_ref[1::2, :]) * 0.25
        s2_ref[...] = jnp.dot(img_ref[0, c], _pool_matrix(512, 4),
                              preferred_element_type=jnp.float32)
        p2 = (s2_ref[0::4, :] + s2_ref[1::4, :]
              + s2_ref[2::4, :] + s2_ref[3::4, :]) * 0.0625
        cols.append(jnp.concatenate(
            [_tokens(p0), _ring(_tokens(p1)), _ring(_tokens(p2))], axis=0))
    out_ref[0] = jnp.concatenate(cols, axis=1)


def kernel(images):
    B, C, H, W = images.shape
    out = pl.pallas_call(
        _fov_kernel,
        grid=(B,),
        in_specs=[pl.BlockSpec((1, C, H, W), lambda b: (b, 0, 0, 0))],
        out_specs=pl.BlockSpec((1, 160, C * 256), lambda b: (b, 0, 0)),
        out_shape=jax.ShapeDtypeStruct((B, 160, C * 256), jnp.float32),
        scratch_shapes=[pltpu.VMEM((256, 128), jnp.float32),
                        pltpu.VMEM((512, 128), jnp.float32)],
        compiler_params=pltpu.CompilerParams(
            dimension_semantics=("arbitrary",)),
    )(images)
    return out.reshape(B, 160, C, _TOK, _TOK)


# speedup vs baseline: 157.4375x; 1.0235x over previous
import jax
import jax.numpy as jnp
from jax.experimental import pallas as pl
from jax.experimental.pallas import tpu as pltpu

_TOK = 16


def _pool_matrix(rows, pool):
    k = jax.lax.broadcasted_iota(jnp.int32, (rows, rows // pool), 0)
    v = jax.lax.broadcasted_iota(jnp.int32, (rows, rows // pool), 1)
    return jnp.where(k // pool == v, 1.0, 0.0).astype(jnp.float32)


def _roll_sub(x, m):
    r = x.reshape(16, 8, 128)
    r = jnp.concatenate([r[:, 8 - m:], r[:, :8 - m]], axis=1)
    return r.reshape(128, 128)


def _gran_xpose(x):
    s = jax.lax.broadcasted_iota(jnp.int32, (128, 128), 0) % 8
    g = jax.lax.broadcasted_iota(jnp.int32, (128, 128), 1) // _TOK
    for m in (4, 2, 1):
        xa = jnp.roll(_roll_sub(x, m), -_TOK * m, axis=1)
        xb = jnp.roll(_roll_sub(x, 8 - m), _TOK * m, axis=1)
        sm = (s & m) != 0
        gm = (g & m) != 0
        x = jnp.where(sm == gm, x, jnp.where(sm, xa, xb))
    return x


def _tokens(p):
    q = _gran_xpose(p).reshape(8, 2, 8, 128)
    return jnp.concatenate(
        [q[:, 0].reshape(64, 128), q[:, 1].reshape(64, 128)], axis=1)


def _ring(x):
    parts = [x[0:16]]
    for i in range(2, 6):
        parts.append(x[8 * i:8 * i + 2])
        parts.append(x[8 * i + 6:8 * i + 8])
    parts.append(x[48:64])
    return jnp.concatenate(parts, axis=0)


_IMGS_PER_PROG = 2


def _fov_kernel(img_ref, out_ref, s1_ref, s2_ref):
    for bb in range(_IMGS_PER_PROG):
        cols = []
        for c in range(3):
            p0 = img_ref[bb, c, 192:320, 192:320]
            s1_ref[...] = jnp.dot(img_ref[bb, c, 128:384, 128:384],
                                  _pool_matrix(256, 2),
                                  preferred_element_type=jnp.float32)
            p1 = (s1_ref[0::2, :] + s1_ref[1::2, :]) * 0.25
            s2_ref[...] = jnp.dot(img_ref[bb, c], _pool_matrix(512, 4),
                                  preferred_element_type=jnp.float32)
            p2 = (s2_ref[0::4, :] + s2_ref[1::4, :]
                  + s2_ref[2::4, :] + s2_ref[3::4, :]) * 0.0625
            cols.append(jnp.concatenate(
                [_tokens(p0), _ring(_tokens(p1)), _ring(_tokens(p2))],
                axis=0))
        out_ref[bb] = jnp.concatenate(cols, axis=1)


def kernel(images):
    B, C, H, W = images.shape
    g = _IMGS_PER_PROG
    out = pl.pallas_call(
        _fov_kernel,
        grid=(B // g,),
        in_specs=[pl.BlockSpec((g, C, H, W), lambda b: (b, 0, 0, 0))],
        out_specs=pl.BlockSpec((g, 160, C * 256), lambda b: (b, 0, 0)),
        out_shape=jax.ShapeDtypeStruct((B, 160, C * 256), jnp.float32),
        scratch_shapes=[pltpu.VMEM((256, 128), jnp.float32),
                        pltpu.VMEM((512, 128), jnp.float32)],
        compiler_params=pltpu.CompilerParams(
            dimension_semantics=("arbitrary",)),
    )(images)
    return out.reshape(B, 160, C, _TOK, _TOK)
